# baseline (speedup 1.0000x reference)
import jax
import jax.numpy as jnp
from jax import lax
from jax.experimental import pallas as pl
from jax.experimental.pallas import tpu as pltpu
from jax.experimental.pallas import tpu_sc as plsc

L = 16
NBINS = 256
HVOL = L * NBINS
IN_ROWS = 32
Q_ROWS = 128
N_CHUNKS = Q_ROWS // IN_ROWS
OUT_ROWS = 32
OUT_H = 512
OUT_W = 512
IMGS_PER_CORE = 12
TASKS_PER_TILE = 3
SUBBLKS = Q_ROWS // OUT_ROWS
N_OUT = TASKS_PER_TILE * SUBBLKS


def _body(x_hbm, out_hbm, inbuf, hist16, redbuf, hist, rowvals, rowbuf,
          shared_h, isem0, isem1, osem0, osem1):
    sid = lax.axis_index("s")
    cid = lax.axis_index("c")
    lanes = lax.iota(jnp.int32, L)
    ones = jnp.full((L,), 1.0, jnp.float32)
    zeros = jnp.zeros((L,), jnp.float32)
    isems = (isem0, isem1)
    osems = (osem0, osem1)

    def img_coords(t):
        img_l = t >> 2
        gi = cid * IMGS_PER_CORE + img_l
        return img_l, gi // 3, gi % 3, (t & 3) * Q_ROWS

    for k in range(TASKS_PER_TILE):
        t = TASKS_PER_TILE * sid + k
        _, bi, ci3, row0 = img_coords(t)

        @plsc.parallel_loop(0, HVOL // L, unroll=8)
        def _(i):
            hist16[pl.ds(i * L, L)] = zeros

        def in_start(ch, b):
            pltpu.async_copy(
                x_hbm.at[bi, ci3, pl.ds(row0 + ch * IN_ROWS, IN_ROWS)],
                inbuf.at[b], isems[b])

        def in_wait(ch, b):
            pltpu.make_async_copy(
                x_hbm.at[bi, ci3, pl.ds(row0 + ch * IN_ROWS, IN_ROWS)],
                inbuf.at[b], isems[b]).wait()

        def consume(b, i):
            r = i >> 5
            g = i & 31
            v = inbuf[b, r, pl.ds(g * L, L)]
            bits = lax.bitcast_convert_type(v + 1.0, jnp.int32)
            idx = ((bits >> 11) & 0xFF0) | lanes
            plsc.addupdate_scatter(hist16, [idx], ones)

        in_start(0, 0)

        @pl.loop(0, N_CHUNKS // 2)
        def _(p):
            ch0 = 2 * p
            in_wait(ch0, 0)
            in_start(ch0 + 1, 1)

            @plsc.parallel_loop(0, (IN_ROWS * OUT_W) // L, unroll=8)
            def _(i):
                consume(0, i)

            in_wait(ch0 + 1, 1)

            @pl.when(p < N_CHUNKS // 2 - 1)
            def _():
                in_start(ch0 + 2, 0)

            @plsc.parallel_loop(0, (IN_ROWS * OUT_W) // L, unroll=8)
            def _(i):
                consume(1, i)

        pltpu.sync_copy(hist16, shared_h.at[t])

    plsc.subcore_barrier()

    def out_dst(jj):
        k, j = jj // SUBBLKS, jj % SUBBLKS
        _, bi, ci3, row0 = img_coords(TASKS_PER_TILE * sid + k)
        return out_hbm.at[bi, ci3, pl.ds(row0 + j * OUT_ROWS, OUT_ROWS)]

    def out_wait(jj, b):
        pltpu.make_async_copy(rowbuf.at[b], out_dst(jj), osems[b]).wait()

    for k in range(TASKS_PER_TILE):
        t = TASKS_PER_TILE * sid + k
        img_l, bi, ci3, row0 = img_coords(t)
        pltpu.sync_copy(shared_h.at[pl.ds(4 * img_l, 4)], redbuf)

        @plsc.parallel_loop(0, HVOL // L, unroll=8)
        def _(i):
            sl = pl.ds(i * L, L)
            hist16[sl] = ((redbuf[0, sl] + redbuf[1, sl])
                          + (redbuf[2, sl] + redbuf[3, sl]))

        for bb in range(6):
            base = jnp.clip((row0 >> 1) - 16 + bb * L, 0, NBINS - L)
            binbase = (lanes + base) * L
            acc = plsc.load_gather(hist16, [binbase])
            for l in range(1, L):
                acc = acc + plsc.load_gather(hist16, [binbase + l])
            hist[pl.ds(base, L)] = acc

        @plsc.parallel_loop(0, Q_ROWS // L)
        def _(g):
            y = (lanes + row0 + g * L).astype(jnp.float32)
            ys = jnp.maximum(y * 0.5 - 0.25, 0.0)
            y0 = ys.astype(jnp.int32)
            wy = ys - y0.astype(jnp.float32)
            y1 = jnp.minimum(y0 + 1, NBINS - 1)
            v0 = plsc.load_gather(hist, [y0])
            v1 = plsc.load_gather(hist, [y1])
            rowvals[pl.ds(g * L, L)] = v0 + wy * (v1 - v0)

        for j in range(SUBBLKS):
            jj = k * SUBBLKS + j
            b = jj & 1
            if jj >= 2:
                out_wait(jj - 2, b)

            @plsc.parallel_loop(0, OUT_ROWS, unroll=2)
            def _(r):
                y = j * OUT_ROWS + r
                v = plsc.load_gather(rowvals, [jnp.zeros((L,), jnp.int32) + y])
                for g in range(OUT_W // L):
                    rowbuf[b, r, pl.ds(g * L, L)] = v

            pltpu.async_copy(rowbuf.at[b], out_dst(jj), osems[b])

    out_wait(N_OUT - 2, 0)
    out_wait(N_OUT - 1, 1)


@jax.jit
def kernel(x):
    b, c, h, w = x.shape

    sc_call = pl.kernel(
        _body,
        out_type=jax.ShapeDtypeStruct((b, 3, h, w), jnp.float32),
        mesh=plsc.VectorSubcoreMesh(core_axis_name="c", subcore_axis_name="s"),
        scratch_types=[
            pltpu.VMEM((2, IN_ROWS, OUT_W), jnp.float32),
            pltpu.VMEM((HVOL,), jnp.float32),
            pltpu.VMEM((4, HVOL), jnp.float32),
            pltpu.VMEM((NBINS,), jnp.float32),
            pltpu.VMEM((Q_ROWS,), jnp.float32),
            pltpu.VMEM((2, OUT_ROWS, OUT_W), jnp.float32),
            pltpu.VMEM_SHARED((48, HVOL), jnp.float32),
            pltpu.SemaphoreType.DMA,
            pltpu.SemaphoreType.DMA,
            pltpu.SemaphoreType.DMA,
            pltpu.SemaphoreType.DMA,
        ],
        compiler_params=pltpu.CompilerParams(needs_layout_passes=False),
    )
    return sc_call(x[:, :3, :, :])


# speedup vs baseline: 1.1656x; 1.1656x over previous
import jax
import jax.numpy as jnp
from jax import lax
from jax.experimental import pallas as pl
from jax.experimental.pallas import tpu as pltpu
from jax.experimental.pallas import tpu_sc as plsc

L = 16
NBC = 24
NBINS = 256
IN_ROWS = 64
N_CHUNKS = 512 // IN_ROWS
ROWS_PER_BLK = 32
N_BLKS = 512 // ROWS_PER_BLK
OUT_H = 512
OUT_W = 512


def _body(x_hbm, out_hbm, inbuf, hist16, hist, rowvals, rowbuf,
          isem0, isem1, osem0, osem1):
    wid = lax.axis_index("s") * 2 + lax.axis_index("c")
    lanes = lax.iota(jnp.int32, L)
    ones = jnp.full((L,), 1.0, jnp.float32)
    zeros = jnp.zeros((L,), jnp.float32)
    isems = (isem0, isem1)
    osems = (osem0, osem1)

    @pl.when(wid < NBC)
    def _():
        bi = wid // 3
        ci = wid % 3

        def in_start(ch, b):
            pltpu.async_copy(x_hbm.at[bi, ci, pl.ds(ch * IN_ROWS, IN_ROWS)],
                             inbuf.at[b], isems[b])

        def in_wait(ch, b):
            pltpu.make_async_copy(
                x_hbm.at[bi, ci, pl.ds(ch * IN_ROWS, IN_ROWS)],
                inbuf.at[b], isems[b]).wait()

        @plsc.parallel_loop(0, (L * NBINS) // L, unroll=8)
        def _(i):
            hist16[pl.ds(i * L, L)] = zeros

        in_start(0, 0)

        def consume(b, i):
            r = i >> 5
            g = i & 31
            v = inbuf[b, r, pl.ds(g * L, L)]
            bits = lax.bitcast_convert_type(v + 1.0, jnp.int32)
            idx = ((bits >> 11) & 0xFF0) | lanes
            plsc.addupdate_scatter(hist16, [idx], ones)

        @pl.loop(0, N_CHUNKS // 2)
        def _(p):
            ch0 = 2 * p
            in_wait(ch0, 0)
            in_start(ch0 + 1, 1)

            @plsc.parallel_loop(0, (IN_ROWS * OUT_W) // L, unroll=8)
            def _(i):
                consume(0, i)

            in_wait(ch0 + 1, 1)

            @pl.when(p < N_CHUNKS // 2 - 1)
            def _():
                in_start(ch0 + 2, 0)

            @plsc.parallel_loop(0, (IN_ROWS * OUT_W) // L, unroll=8)
            def _(i):
                consume(1, i)

        for bb in range(NBINS // L):
            binbase = (lanes + bb * L) * L
            acc = plsc.load_gather(hist16, [binbase])
            for l in range(1, L):
                acc = acc + plsc.load_gather(hist16, [binbase + l])
            hist[pl.ds(bb * L, L)] = acc

        @plsc.parallel_loop(0, OUT_H // L)
        def _(g):
            y = (lanes + g * L).astype(jnp.float32)
            ys = jnp.maximum(y * 0.5 - 0.25, 0.0)
            y0 = ys.astype(jnp.int32)
            wy = ys - y0.astype(jnp.float32)
            y1 = jnp.minimum(y0 + 1, NBINS - 1)
            v0 = plsc.load_gather(hist, [y0])
            v1 = plsc.load_gather(hist, [y1])
            rowvals[pl.ds(g * L, L)] = v0 + wy * (v1 - v0)

        def out_start(blk, b):
            pltpu.async_copy(
                rowbuf.at[b],
                out_hbm.at[bi, ci, pl.ds(blk * ROWS_PER_BLK, ROWS_PER_BLK)],
                osems[b])

        def out_wait(blk, b):
            pltpu.make_async_copy(
                rowbuf.at[b],
                out_hbm.at[bi, ci, pl.ds(blk * ROWS_PER_BLK, ROWS_PER_BLK)],
                osems[b]).wait()

        @pl.loop(0, N_BLKS // 2)
        def _(p):
            for ob in range(2):
                blk = 2 * p + ob

                @pl.when(p > 0)
                def _():
                    out_wait(blk - 2, ob)

                @plsc.parallel_loop(0, ROWS_PER_BLK, unroll=2)
                def _(r):
                    y = blk * ROWS_PER_BLK + r
                    v = plsc.load_gather(
                        rowvals, [jnp.zeros((L,), jnp.int32) + y])
                    for k in range(OUT_W // L):
                        rowbuf[ob, r, pl.ds(k * L, L)] = v

                out_start(blk, ob)

        out_wait(N_BLKS - 2, 0)
        out_wait(N_BLKS - 1, 1)


@jax.jit
def kernel(x):
    b, c, h, w = x.shape

    sc_call = pl.kernel(
        _body,
        out_type=jax.ShapeDtypeStruct((b, 3, h, w), jnp.float32),
        mesh=plsc.VectorSubcoreMesh(core_axis_name="c", subcore_axis_name="s"),
        scratch_types=[
            pltpu.VMEM((2, IN_ROWS, OUT_W), jnp.float32),
            pltpu.VMEM((L * NBINS,), jnp.float32),
            pltpu.VMEM((NBINS,), jnp.float32),
            pltpu.VMEM((OUT_H,), jnp.float32),
            pltpu.VMEM((2, ROWS_PER_BLK, OUT_W), jnp.float32),
            pltpu.SemaphoreType.DMA,
            pltpu.SemaphoreType.DMA,
            pltpu.SemaphoreType.DMA,
            pltpu.SemaphoreType.DMA,
        ],
        compiler_params=pltpu.CompilerParams(needs_layout_passes=False),
    )
    return sc_call(x[:, :3, :, :])
